# baseline (speedup 1.0000x reference)
import functools

import jax
import jax.numpy as jnp
from jax.experimental import pallas as pl
from jax.experimental.pallas import tpu as pltpu

BR = 128


def _monotonic_i32(v):
    s = jax.lax.bitcast_convert_type(v, jnp.int32)
    return jnp.where(s < 0, jnp.bitwise_xor(s, jnp.int32(0x7FFFFFFF)), s)


def _search16(v, k):
    n = v.shape[0]
    one = jnp.ones((), jnp.int16)
    zero = jnp.zeros((), jnp.int16)

    def count_ge(t16):
        c = jnp.where(v >= t16, one, zero)
        mm = c.shape[1]
        while mm > 128 and mm % 2 == 0 and (mm // 2) % 128 == 0:
            mm //= 2
            c = c[:, :mm] + c[:, mm:]
        if mm > 128:
            acc = c[:, :128]
            for j in range(128, mm, 128):
                acc = acc + c[:, j:j + 128]
            c = acc
        return jnp.sum(c.astype(jnp.int32), axis=1, keepdims=True)

    def body(_, carry):
        lo, hi = carry
        mid = lo + ((hi - lo) >> 1)
        pred = count_ge(mid.astype(jnp.int16)) >= k
        return jnp.where(pred, mid, lo), jnp.where(pred, hi, mid)

    lo0 = jnp.full((n, 1), -32768, jnp.int32)
    hi0 = jnp.full((n, 1), 32767, jnp.int32)
    ans, _ = jax.lax.fori_loop(0, 16, body, (lo0, hi0))
    return jnp.where(count_ge(jnp.int16(32767)) >= k, jnp.int32(32767), ans)


def _select_z(pre, k):
    mk = _monotonic_i32(pre)
    k16 = jnp.int16(k)

    hi16 = jax.lax.shift_right_arithmetic(mk, 16).astype(jnp.int16)
    H = _search16(hi16, k16)

    H16 = H.astype(jnp.int16)
    lo16 = jnp.bitwise_xor(mk.astype(jnp.int16), jnp.int16(-0x8000))
    wv = jnp.where(hi16 > H16, jnp.int16(32767),
                   jnp.where(hi16 < H16, jnp.int16(-32768), lo16))
    L = _search16(wv, k16)

    thr = (jax.lax.shift_left(H, 16)
           | (jnp.bitwise_xor(L, jnp.int32(0x8000)) & 0xFFFF))
    return jnp.where(mk >= thr, jnp.maximum(pre, 0.0), 0.0)


def _fused_kernel(x_enc_ref, b_dec_e_ref, W_enc_ref, b_enc_ref,
                  W_dec_ref, b_dec_d_ref, x_loss_ref,
                  z_ref, xhat_ref, loss_ref, pre_ref, *, k, n_blocks):
    s = pl.program_id(0)
    par = jax.lax.rem(s, 2)

    @pl.when(s < n_blocks)
    def _():
        xc = (x_enc_ref[...] - b_dec_e_ref[...]).astype(jnp.bfloat16)
        pre = jax.lax.dot_general(
            xc, W_enc_ref[0], (((1,), (1,)), ((), ())),
            preferred_element_type=jnp.float32) + b_enc_ref[...]
        pre_ref[par] = pre

    @pl.when(s > 0)
    def _():
        pre = pre_ref[1 - par]
        z = _select_z(pre, k)
        z_ref[...] = z
        xh = jax.lax.dot_general(
            z.astype(jnp.bfloat16), W_dec_ref[0], (((1,), (1,)), ((), ())),
            preferred_element_type=jnp.float32) + b_dec_d_ref[...]
        xhat_ref[...] = xh
        r = x_loss_ref[...] - xh

        @pl.when(s == 1)
        def _():
            loss_ref[...] = jnp.zeros((1, 1), jnp.float32)

        loss_ref[...] += jnp.sum(r * r).reshape(1, 1)


def kernel(x, b_dec, W_enc, b_enc, W_dec):
    B, T, D_IN = x.shape
    D_SAE = W_enc.shape[1]
    K = 64
    nb = B // BR
    N = T * nb

    x2 = x.reshape(B, T * D_IN)
    b_dec2 = b_dec.reshape(1, T * D_IN)
    b_enc2 = b_enc.reshape(1, T * D_SAE)

    def enc_i(s):
        se = jnp.minimum(s, N - 1)
        return se % nb, se // nb

    def dec_i(s):
        sd = jnp.maximum(s - 1, 0)
        return sd % nb, sd // nb

    z2, xhat2, loss_sum = pl.pallas_call(
        functools.partial(_fused_kernel, k=K, n_blocks=N),
        grid=(N + 1,),
        in_specs=[
            pl.BlockSpec((BR, D_IN), lambda s: enc_i(s)),
            pl.BlockSpec((1, D_IN), lambda s: (0, enc_i(s)[1])),
            pl.BlockSpec((1, D_SAE, D_IN), lambda s: (enc_i(s)[1], 0, 0)),
            pl.BlockSpec((1, D_SAE), lambda s: (0, enc_i(s)[1])),
            pl.BlockSpec((1, D_IN, D_SAE), lambda s: (dec_i(s)[1], 0, 0)),
            pl.BlockSpec((1, D_IN), lambda s: (0, dec_i(s)[1])),
            pl.BlockSpec((BR, D_IN), lambda s: dec_i(s)),
        ],
        out_specs=[
            pl.BlockSpec((BR, D_SAE), lambda s: dec_i(s)),
            pl.BlockSpec((BR, D_IN), lambda s: dec_i(s)),
            pl.BlockSpec((1, 1), lambda s: (0, 0)),
        ],
        out_shape=[
            jax.ShapeDtypeStruct((B, T * D_SAE), jnp.float32),
            jax.ShapeDtypeStruct((B, T * D_IN), jnp.float32),
            jax.ShapeDtypeStruct((1, 1), jnp.float32),
        ],
        scratch_shapes=[pltpu.VMEM((2, BR, D_SAE), jnp.float32)],
    )(x2, b_dec2, W_enc.astype(jnp.bfloat16), b_enc2,
      W_dec.astype(jnp.bfloat16), b_dec2, x2)

    loss = loss_sum[0, 0] / jnp.float32(B * T)
    return (loss, xhat2.reshape(B, T, D_IN), z2.reshape(B, T, D_SAE))


# speedup vs baseline: 1.1070x; 1.1070x over previous
import functools

import jax
import jax.numpy as jnp
from jax.experimental import pallas as pl
from jax.experimental.pallas import tpu as pltpu

BR = 128


def _monotonic_i32(v):
    s = jax.lax.bitcast_convert_type(v, jnp.int32)
    return jnp.where(s < 0, jnp.bitwise_xor(s, jnp.int32(0x7FFFFFFF)), s)


def _search16(v, k):
    n = v.shape[0]
    one = jnp.ones((), jnp.int16)
    zero = jnp.zeros((), jnp.int16)

    def count_ge(t16):
        c = jnp.where(v >= t16, one, zero)
        mm = c.shape[1]
        while mm > 128 and mm % 2 == 0 and (mm // 2) % 128 == 0:
            mm //= 2
            c = c[:, :mm] + c[:, mm:]
        if mm > 128:
            acc = c[:, :128]
            for j in range(128, mm, 128):
                acc = acc + c[:, j:j + 128]
            c = acc
        return jnp.sum(c.astype(jnp.int32), axis=1, keepdims=True)

    lo = jnp.full((n, 1), -32768, jnp.int32)
    hi = jnp.full((n, 1), 32767, jnp.int32)
    for _ in range(16):
        mid = lo + ((hi - lo) >> 1)
        pred = count_ge(mid.astype(jnp.int16)) >= k
        lo = jnp.where(pred, mid, lo)
        hi = jnp.where(pred, hi, mid)
    return jnp.where(count_ge(jnp.int16(32767)) >= k, jnp.int32(32767), lo)


def _select_z(pre, k):
    mk = _monotonic_i32(pre)
    k16 = jnp.int16(k)

    hi16 = jax.lax.shift_right_arithmetic(mk, 16).astype(jnp.int16)
    H = _search16(hi16, k16)

    H16 = H.astype(jnp.int16)
    lo16 = jnp.bitwise_xor(mk.astype(jnp.int16), jnp.int16(-0x8000))
    wv = jnp.where(hi16 > H16, jnp.int16(32767),
                   jnp.where(hi16 < H16, jnp.int16(-32768), lo16))
    L = _search16(wv, k16)

    thr = (jax.lax.shift_left(H, 16)
           | (jnp.bitwise_xor(L, jnp.int32(0x8000)) & 0xFFFF))
    return jnp.where(mk >= thr, jnp.maximum(pre, 0.0), 0.0)


def _fused_kernel(x_enc_ref, b_dec_e_ref, W_enc_ref, b_enc_ref,
                  W_dec_ref, b_dec_d_ref, x_loss_ref,
                  z_ref, xhat_ref, loss_ref, pre_ref, *, k, n_blocks):
    s = pl.program_id(0)
    par = jax.lax.rem(s, 2)


    xc = (x_enc_ref[...] - b_dec_e_ref[...]).astype(jnp.bfloat16)
    pre_s = jax.lax.dot_general(
        xc, W_enc_ref[0], (((1,), (1,)), ((), ())),
        preferred_element_type=jnp.float32) + b_enc_ref[...]
    pre_ref[par] = pre_s

    pre = pre_ref[1 - par]
    z = _select_z(pre, k)
    z_ref[...] = z
    xh = jax.lax.dot_general(
        z.astype(jnp.bfloat16), W_dec_ref[0], (((1,), (1,)), ((), ())),
        preferred_element_type=jnp.float32) + b_dec_d_ref[...]
    xhat_ref[...] = xh
    r = x_loss_ref[...] - xh

    @pl.when(s == 1)
    def _():
        loss_ref[...] = jnp.zeros((1, 1), jnp.float32)

    loss_ref[...] += jnp.sum(r * r).reshape(1, 1)


def kernel(x, b_dec, W_enc, b_enc, W_dec):
    B, T, D_IN = x.shape
    D_SAE = W_enc.shape[1]
    K = 64
    nb = B // BR
    N = T * nb

    x2 = x.reshape(B, T * D_IN)
    b_dec2 = b_dec.reshape(1, T * D_IN)
    b_enc2 = b_enc.reshape(1, T * D_SAE)

    def enc_i(s):
        se = jnp.minimum(s, N - 1)
        return se % nb, se // nb

    def dec_i(s):
        sd = jnp.maximum(s - 1, 0)
        return sd % nb, sd // nb

    z2, xhat2, loss_sum = pl.pallas_call(
        functools.partial(_fused_kernel, k=K, n_blocks=N),
        grid=(N + 1,),
        in_specs=[
            pl.BlockSpec((BR, D_IN), lambda s: enc_i(s)),
            pl.BlockSpec((1, D_IN), lambda s: (0, enc_i(s)[1])),
            pl.BlockSpec((1, D_SAE, D_IN), lambda s: (enc_i(s)[1], 0, 0)),
            pl.BlockSpec((1, D_SAE), lambda s: (0, enc_i(s)[1])),
            pl.BlockSpec((1, D_IN, D_SAE), lambda s: (dec_i(s)[1], 0, 0)),
            pl.BlockSpec((1, D_IN), lambda s: (0, dec_i(s)[1])),
            pl.BlockSpec((BR, D_IN), lambda s: dec_i(s)),
        ],
        out_specs=[
            pl.BlockSpec((BR, D_SAE), lambda s: dec_i(s)),
            pl.BlockSpec((BR, D_IN), lambda s: dec_i(s)),
            pl.BlockSpec((1, 1), lambda s: (0, 0)),
        ],
        out_shape=[
            jax.ShapeDtypeStruct((B, T * D_SAE), jnp.float32),
            jax.ShapeDtypeStruct((B, T * D_IN), jnp.float32),
            jax.ShapeDtypeStruct((1, 1), jnp.float32),
        ],
        scratch_shapes=[pltpu.VMEM((2, BR, D_SAE), jnp.float32)],
    )(x2, b_dec2, W_enc.astype(jnp.bfloat16), b_enc2,
      W_dec.astype(jnp.bfloat16), b_dec2, x2)

    loss = loss_sum[0, 0] / jnp.float32(B * T)
    return (loss, xhat2.reshape(B, T, D_IN), z2.reshape(B, T, D_SAE))


# speedup vs baseline: 1.1162x; 1.0083x over previous
import functools

import jax
import jax.numpy as jnp
from jax.experimental import pallas as pl
from jax.experimental.pallas import tpu as pltpu

BR = 128


def _monotonic_i32(v):
    s = jax.lax.bitcast_convert_type(v, jnp.int32)
    return jnp.where(s < 0, jnp.bitwise_xor(s, jnp.int32(0x7FFFFFFF)), s)


def _search16(v, k, top_correction=True):
    n = v.shape[0]
    one = jnp.ones((), jnp.int16)
    zero = jnp.zeros((), jnp.int16)

    def count_ge(t16):
        c = jnp.where(v >= t16, one, zero)
        mm = c.shape[1]
        while mm > 128 and mm % 2 == 0 and (mm // 2) % 128 == 0:
            mm //= 2
            c = c[:, :mm] + c[:, mm:]
        if mm > 128:
            acc = c[:, :128]
            for j in range(128, mm, 128):
                acc = acc + c[:, j:j + 128]
            c = acc
        return jnp.sum(c.astype(jnp.int32), axis=1, keepdims=True)

    lo = jnp.full((n, 1), -32768, jnp.int32)
    hi = jnp.full((n, 1), 32767, jnp.int32)
    for _ in range(16):
        mid = lo + ((hi - lo) >> 1)
        pred = count_ge(mid.astype(jnp.int16)) >= k
        lo = jnp.where(pred, mid, lo)
        hi = jnp.where(pred, hi, mid)
    if not top_correction:
        return lo
    return jnp.where(count_ge(jnp.int16(32767)) >= k, jnp.int32(32767), lo)


def _select_z(pre, k):
    nr = pre.shape[0]
    if nr >= 16 and nr % 2 == 0:
        return jnp.concatenate(
            [_select_z_half(pre[: nr // 2], k),
             _select_z_half(pre[nr // 2:], k)], axis=0)
    return _select_z_half(pre, k)


def _select_z_half(pre, k):
    mk = _monotonic_i32(pre)
    k16 = jnp.int16(k)

    hi16 = jax.lax.shift_right_arithmetic(mk, 16).astype(jnp.int16)
    H = _search16(hi16, k16, top_correction=False)

    H16 = H.astype(jnp.int16)
    lo16 = jnp.bitwise_xor(mk.astype(jnp.int16), jnp.int16(-0x8000))
    wv = jnp.where(hi16 > H16, jnp.int16(32767),
                   jnp.where(hi16 < H16, jnp.int16(-32768), lo16))
    L = _search16(wv, k16)

    thr = (jax.lax.shift_left(H, 16)
           | (jnp.bitwise_xor(L, jnp.int32(0x8000)) & 0xFFFF))
    return jnp.where(mk >= thr, jnp.maximum(pre, 0.0), 0.0)


def _fused_kernel(x_enc_ref, b_dec_e_ref, W_enc_ref, b_enc_ref,
                  W_dec_ref, b_dec_d_ref, x_loss_ref,
                  z_ref, xhat_ref, loss_ref, pre_ref, *, k, n_blocks):
    s = pl.program_id(0)
    par = jax.lax.rem(s, 2)


    xc = (x_enc_ref[...] - b_dec_e_ref[...]).astype(jnp.bfloat16)
    pre_s = jax.lax.dot_general(
        xc, W_enc_ref[0], (((1,), (1,)), ((), ())),
        preferred_element_type=jnp.float32) + b_enc_ref[...]
    pre_ref[par] = pre_s

    pre = pre_ref[1 - par]
    z = _select_z(pre, k)
    z_ref[...] = z
    xh = jax.lax.dot_general(
        z.astype(jnp.bfloat16), W_dec_ref[0], (((1,), (1,)), ((), ())),
        preferred_element_type=jnp.float32) + b_dec_d_ref[...]
    xhat_ref[...] = xh
    r = x_loss_ref[...] - xh

    @pl.when(s == 1)
    def _():
        loss_ref[...] = jnp.zeros((1, 1), jnp.float32)

    loss_ref[...] += jnp.sum(r * r).reshape(1, 1)


def kernel(x, b_dec, W_enc, b_enc, W_dec):
    B, T, D_IN = x.shape
    D_SAE = W_enc.shape[1]
    K = 64
    nb = B // BR
    N = T * nb

    x2 = x.reshape(B, T * D_IN)
    b_dec2 = b_dec.reshape(1, T * D_IN)
    b_enc2 = b_enc.reshape(1, T * D_SAE)

    def enc_i(s):
        se = jnp.minimum(s, N - 1)
        return se % nb, se // nb

    def dec_i(s):
        sd = jnp.maximum(s - 1, 0)
        return sd % nb, sd // nb

    z2, xhat2, loss_sum = pl.pallas_call(
        functools.partial(_fused_kernel, k=K, n_blocks=N),
        grid=(N + 1,),
        in_specs=[
            pl.BlockSpec((BR, D_IN), lambda s: enc_i(s)),
            pl.BlockSpec((1, D_IN), lambda s: (0, enc_i(s)[1])),
            pl.BlockSpec((1, D_SAE, D_IN), lambda s: (enc_i(s)[1], 0, 0)),
            pl.BlockSpec((1, D_SAE), lambda s: (0, enc_i(s)[1])),
            pl.BlockSpec((1, D_IN, D_SAE), lambda s: (dec_i(s)[1], 0, 0)),
            pl.BlockSpec((1, D_IN), lambda s: (0, dec_i(s)[1])),
            pl.BlockSpec((BR, D_IN), lambda s: dec_i(s)),
        ],
        out_specs=[
            pl.BlockSpec((BR, D_SAE), lambda s: dec_i(s)),
            pl.BlockSpec((BR, D_IN), lambda s: dec_i(s)),
            pl.BlockSpec((1, 1), lambda s: (0, 0)),
        ],
        out_shape=[
            jax.ShapeDtypeStruct((B, T * D_SAE), jnp.float32),
            jax.ShapeDtypeStruct((B, T * D_IN), jnp.float32),
            jax.ShapeDtypeStruct((1, 1), jnp.float32),
        ],
        scratch_shapes=[pltpu.VMEM((2, BR, D_SAE), jnp.float32)],
    )(x2, b_dec2, W_enc.astype(jnp.bfloat16), b_enc2,
      W_dec.astype(jnp.bfloat16), b_dec2, x2)

    loss = loss_sum[0, 0] / jnp.float32(B * T)
    return (loss, xhat2.reshape(B, T, D_IN), z2.reshape(B, T, D_SAE))
